# baseline (speedup 1.0000x reference)
import functools

import jax
import jax.numpy as jnp
from jax import lax
from jax.experimental import pallas as pl
from jax.experimental.pallas import tpu as pltpu
from jax.experimental.pallas import tpu_sc as plsc

H_DIM = 128
NUM_WORKERS = 32
CHUNK = 128
NBUF = 4


def _gather_kernel(pos_hbm, xp_hbm, table_hbm, out_hbm, pos_v, idx_v, rows_v,
                   psem, gsems, ssems):
    wid = lax.axis_index("s") * 2 + lax.axis_index("c")
    n_chunks = pos_v.shape[0]

    pltpu.sync_copy(pos_hbm.at[wid], pos_v)

    def elem_copy(j):
        return pltpu.make_async_copy(
            xp_hbm.at[pos_v.at[j]], idx_v.at[j], psem
        )

    def fire_elem(j, carry):
        elem_copy(j).start()
        return carry

    def wait_elem(j, carry):
        elem_copy(j).wait()
        return carry

    lax.fori_loop(0, n_chunks, fire_elem, 0)
    lax.fori_loop(0, n_chunks, wait_elem, 0)

    def gather_copy(b, chunk):
        return pltpu.make_async_copy(
            table_hbm.at[idx_v.at[chunk]], rows_v.at[b], gsems.at[b]
        )

    def store_copy(b, chunk):
        base = (wid * n_chunks + chunk) * CHUNK
        return pltpu.make_async_copy(
            rows_v.at[b], out_hbm.at[pl.ds(base, CHUNK)], ssems.at[b]
        )

    for b in range(NBUF):
        gather_copy(b, b).start()

    n_groups = n_chunks // NBUF

    def body(g, carry):
        for b in range(NBUF):
            chunk = g * NBUF + b
            gather_copy(b, chunk).wait()
            store_copy(b, chunk).start()
        for b in range(NBUF):
            chunk = g * NBUF + b
            store_copy(b, chunk).wait()

            @pl.when(g + 1 < n_groups)
            def _():
                gather_copy(b, (g + 1) * NBUF + b).start()

        return carry

    lax.fori_loop(0, n_groups, body, 0)


def kernel(x, table):
    batch, n_fields = x.shape
    total = batch * n_fields
    n_chunks = total // (NUM_WORKERS * CHUNK)

    xp = jnp.pad(x.astype(jnp.int32), ((0, 0), (0, H_DIM - n_fields)))
    xp_flat = xp.reshape(batch * H_DIM)
    k = jnp.arange(total, dtype=jnp.int32)
    pos = (k // n_fields) * H_DIM + k % n_fields
    pos = pos.reshape(NUM_WORKERS, n_chunks, CHUNK)

    mesh = plsc.VectorSubcoreMesh(core_axis_name="c", subcore_axis_name="s")
    run = functools.partial(
        pl.kernel,
        mesh=mesh,
        out_type=jax.ShapeDtypeStruct((total, H_DIM), jnp.float32),
        scratch_types=[
            pltpu.VMEM((n_chunks, CHUNK), jnp.int32),
            pltpu.VMEM((n_chunks, CHUNK), jnp.int32),
            pltpu.VMEM((NBUF, CHUNK, H_DIM), jnp.float32),
            pltpu.SemaphoreType.DMA,
            pltpu.SemaphoreType.DMA((NBUF,)),
            pltpu.SemaphoreType.DMA((NBUF,)),
        ],
    )(_gather_kernel)

    out = run(pos, xp_flat, table)
    return out.reshape(batch, n_fields, H_DIM)


# speedup vs baseline: 1.6399x; 1.6399x over previous
import functools

import jax
import jax.numpy as jnp
from jax import lax
from jax.experimental import pallas as pl
from jax.experimental.pallas import tpu as pltpu
from jax.experimental.pallas import tpu_sc as plsc

H_DIM = 128
N_FIELDS = 26
NUM_WORKERS = 32
BATCH_PER_DMA = 4
CHUNK = BATCH_PER_DMA * N_FIELDS
NBUF = 4


def _gather_kernel(pos_hbm, xp_hbm, table_hbm, out_hbm, pos_v, idx_v, rows_v,
                   psem, gsems, ssems):
    wid = lax.axis_index("s") * 2 + lax.axis_index("c")
    n_idx = pos_v.shape[0]
    n_chunks = n_idx // CHUNK
    rows_per_w = n_idx // N_FIELDS

    pltpu.sync_copy(pos_hbm.at[pl.ds(wid * n_idx, n_idx)], pos_v)

    def elem_copy(j):
        return pltpu.make_async_copy(
            xp_hbm.at[pos_v.at[pl.ds(j * CHUNK, CHUNK)]],
            idx_v.at[pl.ds(j * CHUNK, CHUNK)],
            psem,
        )

    def fire_elem(j, carry):
        elem_copy(j).start()
        return carry

    def wait_elem(j, carry):
        elem_copy(j).wait()
        return carry

    lax.fori_loop(0, n_chunks, fire_elem, 0)
    lax.fori_loop(0, n_chunks, wait_elem, 0)

    def gather_copy(b, chunk):
        return pltpu.make_async_copy(
            table_hbm.at[idx_v.at[pl.ds(chunk * CHUNK, CHUNK)]],
            rows_v.at[b],
            gsems.at[b],
        )

    def store_copy(b, chunk):
        batch0 = wid * rows_per_w + chunk * BATCH_PER_DMA
        return pltpu.make_async_copy(
            rows_v.at[b].reshape(BATCH_PER_DMA, N_FIELDS, H_DIM),
            out_hbm.at[pl.ds(batch0, BATCH_PER_DMA)],
            ssems.at[b],
        )

    for b in range(NBUF):
        gather_copy(b, b).start()

    n_groups = n_chunks // NBUF

    def body(g, carry):
        for b in range(NBUF):
            chunk = g * NBUF + b
            gather_copy(b, chunk).wait()
            store_copy(b, chunk).start()
        for b in range(NBUF):
            chunk = g * NBUF + b
            store_copy(b, chunk).wait()

            @pl.when(g + 1 < n_groups)
            def _():
                gather_copy(b, (g + 1) * NBUF + b).start()

        return carry

    lax.fori_loop(0, n_groups, body, 0)


def kernel(x, table):
    batch, n_fields = x.shape
    total = batch * n_fields

    xp = jnp.pad(x.astype(jnp.int32), ((0, 0), (0, H_DIM - n_fields)))
    xp_flat = xp.reshape(batch * H_DIM)
    k = jnp.arange(total, dtype=jnp.int32)
    pos = (k // n_fields) * H_DIM + k % n_fields

    mesh = plsc.VectorSubcoreMesh(core_axis_name="c", subcore_axis_name="s")
    run = functools.partial(
        pl.kernel,
        mesh=mesh,
        out_type=jax.ShapeDtypeStruct((batch, n_fields, H_DIM), jnp.float32),
        compiler_params=pltpu.CompilerParams(use_tc_tiling_on_sc=True),
        scratch_types=[
            pltpu.VMEM((total // NUM_WORKERS,), jnp.int32),
            pltpu.VMEM((total // NUM_WORKERS,), jnp.int32),
            pltpu.VMEM((NBUF, CHUNK, H_DIM), jnp.float32),
            pltpu.SemaphoreType.DMA,
            pltpu.SemaphoreType.DMA((NBUF,)),
            pltpu.SemaphoreType.DMA((NBUF,)),
        ],
    )(_gather_kernel)

    return run(pos, xp_flat, table)


# speedup vs baseline: 3.6057x; 2.1988x over previous
import functools

import jax
import jax.numpy as jnp
from jax import lax
from jax.experimental import pallas as pl
from jax.experimental.pallas import tpu as pltpu
from jax.experimental.pallas import tpu_sc as plsc

H_DIM = 128
N_FIELDS = 26
F_PAD = 32
NUM_WORKERS = 32
CHUNK = 64
NBUF = 8


def _gather_kernel(xt_hbm, table_hbm, out_hbm, idx_v, rows_v, gsems, ssems):
    wid = lax.axis_index("s") * 2 + lax.axis_index("c")
    rows_per_w = idx_v.shape[1]
    blocks = rows_per_w // CHUNK
    blk_shift = blocks.bit_length() - 1
    n_chunks = N_FIELDS * blocks

    pltpu.sync_copy(xt_hbm.at[:, pl.ds(wid * rows_per_w, rows_per_w)], idx_v)

    def gather_copy(b, chunk):
        f = lax.shift_right_logical(chunk, blk_shift)
        blk = lax.bitwise_and(chunk, blocks - 1)
        return pltpu.make_async_copy(
            table_hbm.at[idx_v.at[f, pl.ds(blk * CHUNK, CHUNK)]],
            rows_v.at[b],
            gsems.at[b],
        )

    def store_copy(b, chunk):
        f = lax.shift_right_logical(chunk, blk_shift)
        blk = lax.bitwise_and(chunk, blocks - 1)
        return pltpu.make_async_copy(
            rows_v.at[b],
            out_hbm.at[f, pl.ds(wid * rows_per_w + blk * CHUNK, CHUNK)],
            ssems.at[b],
        )

    for b in range(NBUF):
        gather_copy(b, b).start()

    n_groups = n_chunks // NBUF

    def body(g, carry):
        for b in range(NBUF):
            chunk = g * NBUF + b
            gather_copy(b, chunk).wait()
            store_copy(b, chunk).start()
        for b in range(NBUF):
            chunk = g * NBUF + b
            store_copy(b, chunk).wait()

            @pl.when(g + 1 < n_groups)
            def _():
                gather_copy(b, (g + 1) * NBUF + b).start()

        return carry

    lax.fori_loop(0, n_groups, body, 0)


def kernel(x, table):
    batch, n_fields = x.shape
    rows_per_w = batch // NUM_WORKERS
    xt = jnp.pad(x.astype(jnp.int32).T, ((0, F_PAD - n_fields), (0, 0)))

    mesh = plsc.VectorSubcoreMesh(core_axis_name="c", subcore_axis_name="s")
    run = functools.partial(
        pl.kernel,
        mesh=mesh,
        out_type=jax.ShapeDtypeStruct((n_fields, batch, H_DIM), jnp.float32),
        scratch_types=[
            pltpu.VMEM((F_PAD, rows_per_w), jnp.int32),
            pltpu.VMEM((NBUF, CHUNK, H_DIM), jnp.float32),
            pltpu.SemaphoreType.DMA((NBUF,)),
            pltpu.SemaphoreType.DMA((NBUF,)),
        ],
    )(_gather_kernel)

    out_fm = run(xt, table)
    return jnp.transpose(out_fm, (1, 0, 2))
